# Initial kernel scaffold; baseline (speedup 1.0000x reference)
import jax
import jax.numpy as jnp
from jax.experimental import pallas as pl


def kernel(x, W, positional_W):
    raise NotImplementedError("write your pallas kernel here")


# speedup vs baseline: 1.1452x; 1.1452x over previous
import functools

import jax
import jax.numpy as jnp
from jax import lax
from jax.experimental import pallas as pl
from jax.experimental.pallas import tpu as pltpu
from jax.experimental.pallas import tpu_sc as plsc

_LANES = 16


@functools.lru_cache(maxsize=None)
def _build(B, S, D, CHUNK):
    N = B * S
    NW = 32
    per_w = N // NW
    assert per_w % CHUNK == 0
    assert per_w % S == 0
    n_chunks = per_w // CHUNK
    chunks_total = N // CHUNK
    groups = D // _LANES

    mesh = plsc.VectorSubcoreMesh(core_axis_name="c", subcore_axis_name="s")

    @functools.partial(
        pl.kernel,
        out_type=jax.ShapeDtypeStruct((N, D), jnp.float32),
        mesh=mesh,
        compiler_params=pltpu.CompilerParams(use_tc_tiling_on_sc=False),
        scratch_types=[
            pltpu.VMEM((per_w,), jnp.int32),
            pltpu.VMEM((S, D), jnp.float32),
            pltpu.VMEM((CHUNK, D), jnp.float32),
            pltpu.SemaphoreType.DMA,
        ],
    )
    def emb(x_hbm, w_hbm, pos_hbm, out_hbm, idx_v, pos_v, buf, sem):
        wid = lax.axis_index("s") * 2 + lax.axis_index("c")
        row_base = wid * per_w

        pltpu.sync_copy(x_hbm.at[pl.ds(row_base, per_w)], idx_v)
        pltpu.sync_copy(pos_hbm.at[pl.ds(0, S)], pos_v)

        def chunk_body(c, p0):
            pltpu.async_copy(
                w_hbm.at[idx_v.at[pl.ds(c * CHUNK, CHUNK)]], buf, sem
            ).wait()

            def row_body(r, p):
                for j in range(groups):
                    sl = pl.ds(j * _LANES, _LANES)
                    buf[r, sl] = buf[r, sl] + pos_v[p, sl]
                pn = p + 1
                return lax.select(pn == S, 0, pn)

            p1 = lax.fori_loop(0, CHUNK, row_body, p0)

            pltpu.sync_copy(buf, out_hbm.at[pl.ds(row_base + c * CHUNK, CHUNK)])
            return p1

        lax.fori_loop(0, n_chunks, chunk_body, 0)

    return emb


def kernel(x, W, positional_W):
    B, S = x.shape
    V, D = W.shape
    CHUNK = 128
    xf = x.reshape(-1).astype(jnp.int32)
    emb = _build(B, S, D, CHUNK)
    out = emb(xf, W, positional_W)
    return out.reshape(B, S, D)


# speedup vs baseline: 1.2831x; 1.1204x over previous
import functools

import jax
import jax.numpy as jnp
from jax import lax
from jax.experimental import pallas as pl
from jax.experimental.pallas import tpu as pltpu
from jax.experimental.pallas import tpu_sc as plsc


@functools.lru_cache(maxsize=None)
def _build(B, S, D, CHUNK):
    N = B * S
    NW = 32
    per_w = N // NW
    assert per_w % CHUNK == 0
    assert per_w % S == 0
    assert S % 8 == 0 and CHUNK % 8 == 0
    n_chunks = per_w // CHUNK

    mesh = plsc.VectorSubcoreMesh(core_axis_name="c", subcore_axis_name="s")

    @functools.partial(
        pl.kernel,
        out_type=jax.ShapeDtypeStruct((N, D), jnp.float32),
        mesh=mesh,
        compiler_params=pltpu.CompilerParams(use_tc_tiling_on_sc=False),
        scratch_types=[
            pltpu.VMEM((per_w,), jnp.int32),
            pltpu.VMEM_SHARED((S + CHUNK, D), jnp.float32),
            pltpu.VMEM((CHUNK, D), jnp.float32),
            pltpu.SemaphoreType.DMA,
        ],
    )
    def emb(x_hbm, w_hbm, pos_hbm, out_hbm, idx_v, pos_sh, buf, sem):
        wid = lax.axis_index("s") * 2 + lax.axis_index("c")
        row_base = wid * per_w

        pltpu.sync_copy(x_hbm.at[pl.ds(row_base, per_w)], idx_v)

        @pl.when(lax.axis_index("s") == 0)
        def _():
            pltpu.sync_copy(pos_hbm.at[pl.ds(0, S)], pos_sh.at[pl.ds(0, S)])
            pltpu.sync_copy(
                pos_hbm.at[pl.ds(0, CHUNK)], pos_sh.at[pl.ds(S, CHUNK)]
            )

        plsc.subcore_barrier()

        for c in range(n_chunks):
            p0 = (c * CHUNK) % S
            pltpu.sync_copy(pos_sh.at[pl.ds(p0, CHUNK)], buf)
            pltpu.async_copy(
                w_hbm.at[idx_v.at[pl.ds(c * CHUNK, CHUNK)]],
                buf,
                sem,
                add=True,
            ).wait()
            pltpu.sync_copy(buf, out_hbm.at[pl.ds(row_base + c * CHUNK, CHUNK)])

    return emb


def kernel(x, W, positional_W):
    B, S = x.shape
    V, D = W.shape
    CHUNK = 128
    xf = x.reshape(-1).astype(jnp.int32)
    emb = _build(B, S, D, CHUNK)
    out = emb(xf, W, positional_W)
    return out.reshape(B, S, D)
